# Initial kernel scaffold; baseline (speedup 1.0000x reference)
import jax
import jax.numpy as jnp
from jax.experimental import pallas as pl


def kernel(x, ln_g, ln_b, light_qkv_w, light_out_w, heavy_gamma, heavy_q_w, heavy_kv_w, heavy_out_w, q_routing_token, kv_routing_token):
    raise NotImplementedError("write your pallas kernel here")


# speedup vs baseline: 1.8247x; 1.8247x over previous
import functools

import numpy as np
import jax
import jax.numpy as jnp
from jax import lax
from jax.experimental import pallas as pl
from jax.experimental.pallas import tpu as pltpu
from jax.experimental.pallas import tpu_sc as plsc

B, N, DIM = 2, 4096, 2048
H, DH = 8, 64
LI = H * DH
K_SEL = 1024
NEG = float(-np.finfo(np.float32).max)
LOGK = float(np.log(np.float32(1152.0)))
SQRT_DIM = float(np.sqrt(DIM))
SCALE = float(DH ** -0.5)


def _ln_qkv_kernel(x_ref, g_ref, b_ref, w_ref, out_ref):
    xb = x_ref[...]
    mu = jnp.mean(xb, axis=-1, keepdims=True)
    xc = xb - mu
    var = jnp.mean(xc * xc, axis=-1, keepdims=True)
    xn = xc / jnp.sqrt(var + 1e-5) * g_ref[...] + b_ref[...]
    out_ref[...] = jnp.dot(xn, w_ref[...], preferred_element_type=jnp.float32)


def _route_kernel(s_ref, u_ref, tn_ref, og_ref, oe_ref):
    s = s_ref[...]

    def it(_, carry):
        a, bb = carry
        sb = s + bb
        m = jnp.max(sb, axis=-1, keepdims=True)
        e = jnp.exp(sb - m)
        ssum = jnp.sum(e, axis=-1, keepdims=True)
        a = LOGK - (jnp.log(ssum) + m)
        bb = -jnp.maximum(s + a, 0.0)
        return a, bb

    a0 = jnp.zeros((4, 1), jnp.float32)
    a, bb = lax.fori_loop(0, 50, it, (a0, -s))
    scores = jnp.exp(s + a + bb)
    u = lax.bitcast_convert_type(scores, jnp.int32)

    def bit_it(i, v):
        trial = v | (jnp.int32(1) << (30 - i))
        cnt = jnp.sum((u >= trial).astype(jnp.int32), axis=-1, keepdims=True)
        return jnp.where(cnt >= K_SEL, trial, v)

    t = lax.fori_loop(0, 31, bit_it, jnp.zeros((4, 1), jnp.int32))
    cnt_gt = jnp.sum((u > t).astype(jnp.int32), axis=-1, keepdims=True)
    need = K_SEL - cnt_gt
    col = lax.broadcasted_iota(jnp.int32, (4, 128), 1)
    u_ref[...] = u
    tn_ref[...] = jnp.where(col == 0, t, 0) + jnp.where(col == 1, need, 0)
    mgt = (u > t).astype(jnp.float32)
    meq = (u == t).astype(jnp.float32)
    gi = lax.broadcasted_iota(jnp.int32, (N, 256), 0)
    gj = lax.broadcasted_iota(jnp.int32, (N, 256), 1)
    G = jnp.where(gi // 16 == gj, 1.0, 0.0).astype(jnp.float32)
    cg = jnp.dot(mgt, G, preferred_element_type=jnp.float32)
    ce = jnp.dot(meq, G, preferred_element_type=jnp.float32)
    li = lax.broadcasted_iota(jnp.int32, (256, 256), 0)
    lj = lax.broadcasted_iota(jnp.int32, (256, 256), 1)
    LT = jnp.where(li < lj, 1.0, 0.0).astype(jnp.float32)
    og_ref[...] = jnp.dot(cg, LT, preferred_element_type=jnp.float32).astype(jnp.int32)
    oe_ref[...] = jnp.dot(ce, LT, preferred_element_type=jnp.float32).astype(jnp.int32)


LC = 512
LB = LC + 128


def _light_attn_kernel(prev_ref, cur_ref, next_ref, w_ref, out_ref):
    pid = pl.program_id(0)
    local_base = (pid % (N // LC)) * LC
    prev = prev_ref[...]
    cur = cur_ref[...]
    nxt = next_ref[...]
    i_idx = lax.broadcasted_iota(jnp.int32, (LC, LB), 0)
    j_idx = lax.broadcasted_iota(jnp.int32, (LC, LB), 1)
    rel = j_idx - (i_idx // 64) * 64
    key_local = local_base - 64 + j_idx
    valid = (rel >= 0) & (rel < 192) & (key_local >= 0) & (key_local < N)
    o_parts = []
    for h in range(H):
        qc, kc, vc = h * DH, LI + h * DH, 2 * LI + h * DH
        q = cur[:, qc:qc + DH] * SCALE
        k_ext = jnp.concatenate(
            [prev[LC - 64:LC, kc:kc + DH], cur[:, kc:kc + DH], nxt[0:64, kc:kc + DH]], axis=0)
        v_ext = jnp.concatenate(
            [prev[LC - 64:LC, vc:vc + DH], cur[:, vc:vc + DH], nxt[0:64, vc:vc + DH]], axis=0)
        sim = lax.dot_general(q, k_ext, (((1,), (1,)), ((), ())),
                              preferred_element_type=jnp.float32)
        sim = jnp.where(valid, sim, NEG)
        m = jnp.max(sim, axis=-1, keepdims=True)
        p = jnp.exp(sim - m)
        attn = p / jnp.sum(p, axis=-1, keepdims=True)
        o_parts.append(lax.dot_general(attn, v_ext, (((1,), (0,)), ((), ())),
                                       preferred_element_type=jnp.float32))
    o = jnp.concatenate(o_parts, axis=1)
    out_ref[...] = jnp.dot(o, w_ref[...], preferred_element_type=jnp.float32)


def _heavy_proj_kernel(x_ref, w_ref, g_ref, out_ref):
    xb = x_ref[0]
    nrm = jnp.sqrt(jnp.sum(xb * xb, axis=-1, keepdims=True))
    xn = xb / jnp.maximum(nrm, 1e-12) * SQRT_DIM * g_ref[...]
    out_ref[0] = jnp.dot(xn, w_ref[...], preferred_element_type=jnp.float32)


def _heavy_attn_kernel(h_ref, w_ref, out_ref):
    hb = h_ref[0]
    o_parts = []
    for h in range(H):
        q = hb[:, h * DH:(h + 1) * DH] * SCALE
        k = hb[:, LI + h * 128: LI + h * 128 + DH]
        v = hb[:, LI + h * 128 + DH: LI + h * 128 + 128]
        sim = lax.dot_general(q, k, (((1,), (1,)), ((), ())),
                              preferred_element_type=jnp.float32)
        m = jnp.max(sim, axis=-1, keepdims=True)
        p = jnp.exp(sim - m)
        attn = p / jnp.sum(p, axis=-1, keepdims=True)
        o_parts.append(lax.dot_general(attn, v, (((1,), (0,)), ((), ())),
                                       preferred_element_type=jnp.float32))
    o = jnp.concatenate(o_parts, axis=1)
    out_ref[0] = jnp.dot(o, w_ref[...], preferred_element_type=jnp.float32)


def _wid():
    return lax.axis_index("s") * 2 + lax.axis_index("c")


def _sc_compact_body(u_hbm, tn_hbm, og_hbm, oe_hbm, idx_hbm,
                     u_v, idx_v, eq_v, tn_v, og_v, oe_v):
    wid = _wid()

    @pl.when(wid < 4)
    def _():
        row = wid
        pltpu.sync_copy(u_hbm.at[row], u_v)
        pltpu.sync_copy(tn_hbm.at[row], tn_v)
        pltpu.sync_copy(og_hbm.at[row], og_v)
        pltpu.sync_copy(oe_hbm.at[row], oe_v)
        tnv = tn_v[pl.ds(0, 16)]
        t = tnv[0]
        need = tnv[1]
        nbase = jnp.int32(K_SEL) - need
        gbase = (row & 1) * jnp.int32(N)
        lanes = lax.iota(jnp.int32, 16)
        big = jnp.int32(1 << 30)

        def outer(j16, _):
            offg16 = og_v[pl.ds(j16 * 16, 16)]
            offe16 = oe_v[pl.ds(j16 * 16, 16)]
            for lane in range(16):
                tok = j16 * 256 + lane * 16
                v16 = u_v[pl.ds(tok, 16)]
                ii = lanes + tok
                m_gt = v16 > t
                m_eq = v16 == t
                sg = lax.sort(jnp.where(m_gt, ii, big))
                se = lax.sort(jnp.where(m_eq, ii, big))
                idx_v[pl.ds(offg16[lane], 16)] = sg + gbase
                eq_v[pl.ds(offe16[lane], 16)] = se + gbase
            return 0

        lax.fori_loop(0, 16, outer, 0)

        def cbody(j, _):
            idx_v[pl.ds(nbase + j * 16, 16)] = eq_v[pl.ds(j * 16, 16)]
            return 0

        lax.fori_loop(0, (need + 15) // 16, cbody, 0)
        pltpu.sync_copy(idx_v.at[pl.ds(0, K_SEL)], idx_hbm.at[row])


def _sc_gather_body(x_hbm, idx_hbm, routed_hbm,
                    idx_a, idx_b, rows_a, rows_b,
                    gsem_a, gsem_b, osem_a, osem_b):
    wid = _wid()
    idxs = (idx_a, idx_b)
    rows = (rows_a, rows_b)
    gsems = (gsem_a, gsem_b)
    osems = (osem_a, osem_b)

    def issue(i):
        p = i % 2
        r = i // 2
        base = wid * 32 + (i % 2) * 16
        pltpu.sync_copy(idx_hbm.at[r, pl.ds(base, 16)], idxs[p])
        return pltpu.async_copy(x_hbm.at[idxs[p]], rows[p], gsems[p])

    gcp = [issue(0), None]
    ocp = [None, None]
    for i in range(8):
        p = i % 2
        if i + 1 < 8:
            q = (i + 1) % 2
            if ocp[q] is not None:
                ocp[q].wait()
            gcp[q] = issue(i + 1)
        gcp[p].wait()
        r = i // 2
        base = wid * 32 + (i % 2) * 16
        ocp[p] = pltpu.async_copy(rows[p], routed_hbm.at[r, pl.ds(base, 16)],
                                  osems[p])
    ocp[0].wait()
    ocp[1].wait()


_SROWS = 8


def _sc_scatter_add_body(lo_ref, ro_hbm, idx_hbm,
                         idx_a, idx_b, acc_a, acc_b, r_a, r_b,
                         gsem_a, gsem_b, rsem_a, rsem_b, ssem_a, ssem_b):
    wid = _wid()
    idxs = (idx_a, idx_b)
    accs = (acc_a, acc_b)
    rvs = (r_a, r_b)
    gsems = (gsem_a, gsem_b)
    rsems = (rsem_a, rsem_b)
    ssems = (ssem_a, ssem_b)
    nchunk = 64 // _SROWS

    def issue(i):
        p = i % 2
        base = wid * 64 + i * _SROWS
        pltpu.sync_copy(idx_hbm.at[pl.ds(base, _SROWS)], idxs[p])
        g = pltpu.async_copy(lo_ref.at[idxs[p]], accs[p], gsems[p])
        r = pltpu.async_copy(ro_hbm.at[pl.ds(base, _SROWS)], rvs[p], rsems[p])
        return g, r

    gcp = [None, None]
    scp = [None, None]
    gcp[0] = issue(0)
    for i in range(nchunk):
        p = i % 2
        if i + 1 < nchunk:
            q = (i + 1) % 2
            if scp[q] is not None:
                scp[q].wait()
            gcp[q] = issue(i + 1)
        gcp[p][0].wait()
        gcp[p][1].wait()
        acc_v, r_v = accs[p], rvs[p]

        @pl.loop(0, _SROWS)
        def _(i2):
            @pl.loop(0, DIM // 16, unroll=8)
            def _(j):
                sl = pl.ds(j * 16, 16)
                acc_v[i2, sl] = acc_v[i2, sl] + r_v[i2, sl]

        scp[p] = pltpu.async_copy(acc_v, lo_ref.at[idxs[p]], ssems[p])
    scp[0].wait()
    scp[1].wait()


@functools.lru_cache(maxsize=1)
def _sc_kernels():
    mesh = plsc.VectorSubcoreMesh(
        core_axis_name="c", subcore_axis_name="s", num_cores=2, num_subcores=16)
    cp = pltpu.CompilerParams(needs_layout_passes=False)
    compact = pl.kernel(
        _sc_compact_body,
        out_type=jax.ShapeDtypeStruct((4, K_SEL), jnp.int32),
        mesh=mesh,
        scratch_types=[
            pltpu.VMEM((N,), jnp.int32),
            pltpu.VMEM((K_SEL + 16,), jnp.int32),
            pltpu.VMEM((N + 16,), jnp.int32),
            pltpu.VMEM((128,), jnp.int32),
            pltpu.VMEM((256,), jnp.int32),
            pltpu.VMEM((256,), jnp.int32),
        ],
        compiler_params=cp,
    )
    gather = pl.kernel(
        _sc_gather_body,
        out_type=jax.ShapeDtypeStruct((4, K_SEL, DIM), jnp.float32),
        mesh=mesh,
        scratch_types=[
            pltpu.VMEM((16,), jnp.int32),
            pltpu.VMEM((16,), jnp.int32),
            pltpu.VMEM((16, DIM), jnp.float32),
            pltpu.VMEM((16, DIM), jnp.float32),
            pltpu.SemaphoreType.DMA,
            pltpu.SemaphoreType.DMA,
            pltpu.SemaphoreType.DMA,
            pltpu.SemaphoreType.DMA,
        ],
        compiler_params=cp,
    )
    scatter_add = pl.kernel(
        _sc_scatter_add_body,
        out_type=(),
        mesh=mesh,
        scratch_types=[
            pltpu.VMEM((_SROWS,), jnp.int32),
            pltpu.VMEM((_SROWS,), jnp.int32),
            pltpu.VMEM((_SROWS, DIM), jnp.float32),
            pltpu.VMEM((_SROWS, DIM), jnp.float32),
            pltpu.VMEM((_SROWS, DIM), jnp.float32),
            pltpu.VMEM((_SROWS, DIM), jnp.float32),
            pltpu.SemaphoreType.DMA,
            pltpu.SemaphoreType.DMA,
            pltpu.SemaphoreType.DMA,
            pltpu.SemaphoreType.DMA,
            pltpu.SemaphoreType.DMA,
            pltpu.SemaphoreType.DMA,
        ],
        compiler_params=cp,
    )
    return compact, gather, scatter_add


def kernel(x, ln_g, ln_b, light_qkv_w, light_out_w, heavy_gamma, heavy_q_w,
           heavy_kv_w, heavy_out_w, q_routing_token, kv_routing_token):
    f32 = jnp.float32
    x2 = x.reshape(B * N, DIM)

    s_q = jnp.einsum('bnd,rd->brn', x, q_routing_token).reshape(B, N)
    s_kv = jnp.einsum('bnd,rd->brn', x, kv_routing_token).reshape(B, N)
    s4 = jnp.concatenate([s_q, s_kv], axis=0)

    qkv = pl.pallas_call(
        _ln_qkv_kernel,
        grid=(16,),
        in_specs=[
            pl.BlockSpec((512, DIM), lambda i: (i, 0)),
            pl.BlockSpec((1, DIM), lambda i: (0, 0)),
            pl.BlockSpec((1, DIM), lambda i: (0, 0)),
            pl.BlockSpec((DIM, 3 * LI), lambda i: (0, 0)),
        ],
        out_specs=pl.BlockSpec((512, 3 * LI), lambda i: (i, 0)),
        out_shape=jax.ShapeDtypeStruct((B * N, 3 * LI), f32),
    )(x2, ln_g.reshape(1, DIM), ln_b.reshape(1, DIM), light_qkv_w)

    u4, tn, og, oe = pl.pallas_call(
        _route_kernel,
        out_shape=[
            jax.ShapeDtypeStruct((4, N), jnp.int32),
            jax.ShapeDtypeStruct((4, 128), jnp.int32),
            jax.ShapeDtypeStruct((4, 256), jnp.int32),
            jax.ShapeDtypeStruct((4, 256), jnp.int32),
        ],
    )(s4)

    sc_compact, sc_gather, sc_scatter_add = _sc_kernels()
    idx4 = sc_compact(u4, tn, og, oe)

    routed = sc_gather(x2, idx4)

    w_heavy = jnp.concatenate([heavy_q_w, heavy_kv_w.T], axis=1)
    hqkv = pl.pallas_call(
        _heavy_proj_kernel,
        grid=(B, 3),
        in_specs=[
            pl.BlockSpec((1, K_SEL, DIM), lambda b, j: (b + 2 * jnp.minimum(j, 1), 0, 0)),
            pl.BlockSpec((DIM, LI), lambda b, j: (0, j)),
            pl.BlockSpec((1, DIM), lambda b, j: (0, 0)),
        ],
        out_specs=pl.BlockSpec((1, K_SEL, LI), lambda b, j: (b, 0, j)),
        out_shape=jax.ShapeDtypeStruct((B, K_SEL, 3 * LI), f32),
    )(routed, w_heavy, heavy_gamma.reshape(1, DIM))

    routed_out = pl.pallas_call(
        _heavy_attn_kernel,
        grid=(B,),
        in_specs=[
            pl.BlockSpec((1, K_SEL, 3 * LI), lambda b: (b, 0, 0)),
            pl.BlockSpec((LI, DIM), lambda b: (0, 0)),
        ],
        out_specs=pl.BlockSpec((1, K_SEL, DIM), lambda b: (b, 0, 0)),
        out_shape=jax.ShapeDtypeStruct((B, K_SEL, DIM), f32),
    )(hqkv, heavy_out_w)

    n_lc = B * N // LC
    light_out = pl.pallas_call(
        _light_attn_kernel,
        grid=(n_lc,),
        in_specs=[
            pl.BlockSpec((LC, 3 * LI), lambda i: (jnp.maximum(i - 1, 0), 0)),
            pl.BlockSpec((LC, 3 * LI), lambda i: (i, 0)),
            pl.BlockSpec((LC, 3 * LI), lambda i: (jnp.minimum(i + 1, n_lc - 1), 0)),
            pl.BlockSpec((LI, DIM), lambda i: (0, 0)),
        ],
        out_specs=pl.BlockSpec((LC, DIM), lambda i: (i, 0)),
        out_shape=jax.ShapeDtypeStruct((B * N, DIM), f32),
    )(qkv, qkv, qkv, light_out_w)

    lo_ref = jax.new_ref(light_out)
    sc_scatter_add(lo_ref, routed_out.reshape(B * K_SEL, DIM),
                   idx4[:2].reshape(B * K_SEL))
    return lo_ref[...].reshape(B, N, DIM)
